# baseline (speedup 1.0000x reference)
import functools

import jax
import jax.numpy as jnp
from jax import lax
from jax.experimental import pallas as pl
from jax.experimental.pallas import tpu as pltpu
from jax.experimental.pallas import tpu_sc as plsc

NC = 2
NS = 16
NW = NC * NS
B = 16384
D = 32
BPW = B // NW

_vector_mesh = plsc.VectorSubcoreMesh(
    core_axis_name="c", subcore_axis_name="s", num_cores=NC, num_subcores=NS
)


def _gather_kernel_body(
    utab_hbm, itab_hbm, uidx_hbm, iidx_hbm,
    urows_hbm, irows_hbm,
    uidx_v, iidx_v, urows_v, irows_v, sem_u, sem_i,
):
    wid = lax.axis_index("s") * NC + lax.axis_index("c")
    base = wid * BPW
    pltpu.sync_copy(uidx_hbm.at[pl.ds(base, BPW)], uidx_v)
    pltpu.sync_copy(iidx_hbm.at[pl.ds(base, BPW)], iidx_v)
    cu = pltpu.async_copy(utab_hbm.at[uidx_v], urows_v, sem_u)
    ci = pltpu.async_copy(itab_hbm.at[iidx_v], irows_v, sem_i)
    cu.wait()
    ci.wait()
    pltpu.sync_copy(urows_v, urows_hbm.at[pl.ds(base, BPW)])
    pltpu.sync_copy(irows_v, irows_hbm.at[pl.ds(base, BPW)])


@jax.jit
def _sc_gather(user_table, item_table, userinput, iteminput):
    rows = jax.ShapeDtypeStruct((B, D), jnp.float32)
    k = pl.kernel(
        _gather_kernel_body,
        out_type=(rows, rows),
        mesh=_vector_mesh,
        scratch_types=[
            pltpu.VMEM((BPW,), jnp.int32),
            pltpu.VMEM((BPW,), jnp.int32),
            pltpu.VMEM((BPW, D), jnp.float32),
            pltpu.VMEM((BPW, D), jnp.float32),
            pltpu.SemaphoreType.DMA,
            pltpu.SemaphoreType.DMA,
        ],
        compiler_params=pltpu.CompilerParams(use_tc_tiling_on_sc=False),
    )
    return k(user_table, item_table, userinput, iteminput)


def _head_kernel_body(u_ref, i_ref, w_ref, b_ref, o_ref):
    p = u_ref[...] * i_ref[...]
    s = jnp.sum(p * w_ref[...], axis=1) + b_ref[0]
    o_ref[...] = jax.nn.sigmoid(s)


@jax.jit
def _tc_head(urows, irows, W, b):
    return pl.pallas_call(
        _head_kernel_body,
        out_shape=jax.ShapeDtypeStruct((B,), jnp.float32),
    )(urows, irows, W, b)


def kernel(userinput, iteminput, user_table, item_table, W, b):
    urows, irows = _sc_gather(
        user_table, item_table,
        userinput.astype(jnp.int32), iteminput.astype(jnp.int32),
    )
    return _tc_head(urows, irows, W, b)
